# baseline (speedup 1.0000x reference)
import functools

import jax
import jax.numpy as jnp
from jax import lax
from jax.experimental import pallas as pl
from jax.experimental.pallas import tpu as pltpu
from jax.experimental.pallas import tpu_sc as plsc

_PAD = 0
_NEG = -99999.0
_BB = 256


def _body(ps_ref, act_ref, u_ref,
          logits_ref, loss_ref, aid_ref, ent_o_ref, fidx_ref,
          ent_s, sem):
    n = ps_ref.shape[0]
    i = pl.program_id(0)
    cp = pltpu.make_async_copy(
        act_ref.at[:, 1, pl.ds(i * _BB, _BB)], ent_s, sem)
    cp.start()
    ps = ps_ref[...]
    u = u_ref[...]
    gumbel = -jnp.log(-jnp.log(u))
    cp.wait()
    ent = ent_s[...]

    scores = jnp.where(ent == _PAD, _NEG, ps)
    m = jnp.max(scores, axis=0, keepdims=True)
    shifted = scores - m
    sumexp = jnp.sum(jnp.exp(shifted), axis=0, keepdims=True)
    logits = shifted - jnp.log(sumexp)
    logits_ref[...] = logits

    y = logits + gumbel
    ymax = jnp.max(y, axis=0, keepdims=True)
    n_iota = jax.lax.broadcasted_iota(jnp.int32, y.shape, 0)
    idx = jnp.min(jnp.where(y == ymax, n_iota, jnp.int32(n)),
                  axis=0, keepdims=True)
    aid_ref[...] = idx

    sel = n_iota == idx
    loss_ref[...] = -jnp.sum(jnp.where(sel, logits, 0.0), axis=0, keepdims=True)
    ent_o_ref[...] = jnp.sum(jnp.where(sel, ent, 0), axis=0, keepdims=True)

    gb = i * _BB + jax.lax.broadcasted_iota(jnp.int32, (1, _BB), 1)
    fidx_ref[...] = idx * 32768 + (gb // 128) * 256 + (gb % 128)


def _rel_gather(B):
    mesh = plsc.VectorSubcoreMesh(core_axis_name="c", subcore_axis_name="s")
    rows = B // 128
    rows_per_w = rows // 32

    @functools.partial(
        pl.kernel, mesh=mesh,
        out_type=jax.ShapeDtypeStruct((rows, 128), jnp.int32),
        scratch_types=[
            pltpu.VMEM((128,), jnp.int32),
            pltpu.VMEM((128,), jnp.int32),
            pltpu.SemaphoreType.DMA,
        ],
    )
    def k(flat_hbm, fidx_hbm, out_hbm, idx_v, rel_v, sem):
        wid = lax.axis_index("s") * 2 + lax.axis_index("c")
        for j in range(rows_per_w):
            r = wid * rows_per_w + j
            pltpu.sync_copy(fidx_hbm.at[r], idx_v)
            pltpu.async_copy(flat_hbm.at[idx_v], rel_v, sem).wait()
            pltpu.sync_copy(rel_v, out_hbm.at[r])

    return k


def kernel(prelim_scores, actions_id, u):
    B, N = prelim_scores.shape
    ps_t = prelim_scores.T
    u_t = u.T
    acts_t = jnp.transpose(actions_id, (1, 2, 0))

    col_spec = pl.BlockSpec((N, _BB), lambda i: (0, i))
    act_spec = pl.BlockSpec(memory_space=pltpu.MemorySpace.HBM)
    one_spec = pl.BlockSpec((1, _BB), lambda i: (0, i))
    outs = pl.pallas_call(
        _body,
        grid=(B // _BB,),
        in_specs=[col_spec, act_spec, col_spec],
        out_specs=[col_spec, one_spec, one_spec, one_spec, one_spec],
        out_shape=[
            jax.ShapeDtypeStruct((N, B), jnp.float32),
            jax.ShapeDtypeStruct((1, B), jnp.float32),
            jax.ShapeDtypeStruct((1, B), jnp.int32),
            jax.ShapeDtypeStruct((1, B), jnp.int32),
            jax.ShapeDtypeStruct((1, B), jnp.int32),
        ],
        scratch_shapes=[
            pltpu.VMEM((N, _BB), jnp.int32),
            pltpu.SemaphoreType.DMA,
        ],
    )(ps_t, acts_t, u_t)
    logits_t, loss, aid, ent_o, fidx = outs

    flat = jnp.transpose(
        actions_id.reshape(B // 128, 128, N, 2), (2, 0, 3, 1)
    ).reshape(N * B * 2)
    fidx2 = fidx.reshape(B // 128, 128)
    rel2 = _rel_gather(B)(flat, fidx2)
    return (loss[0], logits_t.T, aid[0], ent_o[0], rel2.reshape(B))


# speedup vs baseline: 1.2308x; 1.2308x over previous
import functools

import jax
import jax.numpy as jnp
from jax import lax
from jax.experimental import pallas as pl
from jax.experimental.pallas import tpu as pltpu
from jax.experimental.pallas import tpu_sc as plsc

_PAD = 0
_NEG = -99999.0
_BB = 512


def _body(ps_ref, act_ref, u_ref,
          logits_ref, loss_ref, aid_ref, ent_o_ref, fidx_ref,
          ent_s, sem):
    n = ps_ref.shape[0]
    i = pl.program_id(0)
    cp = pltpu.make_async_copy(
        act_ref.at[:, 1, pl.ds(i * _BB, _BB)], ent_s, sem)
    cp.start()
    ps = ps_ref[...]
    u = u_ref[...]
    gumbel = -jnp.log(-jnp.log(u))
    cp.wait()
    ent = ent_s[...]

    scores = jnp.where(ent == _PAD, _NEG, ps)
    m = jnp.max(scores, axis=0, keepdims=True)
    shifted = scores - m
    sumexp = jnp.sum(jnp.exp(shifted), axis=0, keepdims=True)
    logits = shifted - jnp.log(sumexp)
    logits_ref[...] = logits

    y = logits + gumbel
    ymax = jnp.max(y, axis=0, keepdims=True)
    n_iota = jax.lax.broadcasted_iota(jnp.int32, y.shape, 0)
    idx = jnp.min(jnp.where(y == ymax, n_iota, jnp.int32(n)),
                  axis=0, keepdims=True)
    aid_ref[...] = idx

    sel = n_iota == idx
    loss_ref[...] = -jnp.sum(jnp.where(sel, logits, 0.0), axis=0, keepdims=True)
    ent_o_ref[...] = jnp.sum(jnp.where(sel, ent, 0), axis=0, keepdims=True)

    gb = i * _BB + jax.lax.broadcasted_iota(jnp.int32, (1, _BB), 1)
    fidx_ref[...] = idx * 32768 + (gb // 128) * 256 + (gb % 128)


def _rel_gather(B):
    mesh = plsc.VectorSubcoreMesh(core_axis_name="c", subcore_axis_name="s")
    rows = B // 128
    rows_per_w = rows // 32

    @functools.partial(
        pl.kernel, mesh=mesh,
        out_type=jax.ShapeDtypeStruct((rows, 128), jnp.int32),
        scratch_types=[
            pltpu.VMEM((128,), jnp.int32),
            pltpu.VMEM((128,), jnp.int32),
            pltpu.SemaphoreType.DMA,
        ],
    )
    def k(flat_hbm, fidx_hbm, out_hbm, idx_v, rel_v, sem):
        wid = lax.axis_index("s") * 2 + lax.axis_index("c")
        for j in range(rows_per_w):
            r = wid * rows_per_w + j
            pltpu.sync_copy(fidx_hbm.at[r], idx_v)
            pltpu.async_copy(flat_hbm.at[idx_v], rel_v, sem).wait()
            pltpu.sync_copy(rel_v, out_hbm.at[r])

    return k


def kernel(prelim_scores, actions_id, u):
    B, N = prelim_scores.shape
    ps_t = prelim_scores.T
    u_t = u.T
    acts_t = jnp.transpose(actions_id, (1, 2, 0))

    col_spec = pl.BlockSpec((N, _BB), lambda i: (0, i))
    act_spec = pl.BlockSpec(memory_space=pltpu.MemorySpace.HBM)
    one_spec = pl.BlockSpec((1, _BB), lambda i: (0, i))
    outs = pl.pallas_call(
        _body,
        grid=(B // _BB,),
        in_specs=[col_spec, act_spec, col_spec],
        out_specs=[col_spec, one_spec, one_spec, one_spec, one_spec],
        out_shape=[
            jax.ShapeDtypeStruct((N, B), jnp.float32),
            jax.ShapeDtypeStruct((1, B), jnp.float32),
            jax.ShapeDtypeStruct((1, B), jnp.int32),
            jax.ShapeDtypeStruct((1, B), jnp.int32),
            jax.ShapeDtypeStruct((1, B), jnp.int32),
        ],
        scratch_shapes=[
            pltpu.VMEM((N, _BB), jnp.int32),
            pltpu.SemaphoreType.DMA,
        ],
    )(ps_t, acts_t, u_t)
    logits_t, loss, aid, ent_o, fidx = outs

    flat = jnp.transpose(
        actions_id.reshape(B // 128, 128, N, 2), (2, 0, 3, 1)
    ).reshape(N * B * 2)
    fidx2 = fidx.reshape(B // 128, 128)
    rel2 = _rel_gather(B)(flat, fidx2)
    return (loss[0], logits_t.T, aid[0], ent_o[0], rel2.reshape(B))


# speedup vs baseline: 1.3938x; 1.1325x over previous
import functools

import jax
import jax.numpy as jnp
from jax import lax
from jax.experimental import pallas as pl
from jax.experimental.pallas import tpu as pltpu
from jax.experimental.pallas import tpu_sc as plsc

_PAD = 0
_NEG = -99999.0
_BB = 1024


def _body(ps_ref, act_ref, u_ref,
          logits_ref, loss_ref, aid_ref, ent_o_ref, fidx_ref,
          ent_s, sem):
    n = ps_ref.shape[0]
    i = pl.program_id(0)
    cp = pltpu.make_async_copy(
        act_ref.at[:, 1, pl.ds(i * _BB, _BB)], ent_s, sem)
    cp.start()
    ps = ps_ref[...]
    u = u_ref[...]
    gumbel = -jnp.log(-jnp.log(u))
    cp.wait()
    ent = ent_s[...]

    scores = jnp.where(ent == _PAD, _NEG, ps)
    m = jnp.max(scores, axis=0, keepdims=True)
    shifted = scores - m
    sumexp = jnp.sum(jnp.exp(shifted), axis=0, keepdims=True)
    logits = shifted - jnp.log(sumexp)
    logits_ref[...] = logits

    y = logits + gumbel
    ymax = jnp.max(y, axis=0, keepdims=True)
    n_iota = jax.lax.broadcasted_iota(jnp.int32, y.shape, 0)
    idx = jnp.min(jnp.where(y == ymax, n_iota, jnp.int32(n)),
                  axis=0, keepdims=True)
    aid_ref[...] = idx

    sel = n_iota == idx
    loss_ref[...] = -jnp.sum(jnp.where(sel, logits, 0.0), axis=0, keepdims=True)
    ent_o_ref[...] = jnp.sum(jnp.where(sel, ent, 0), axis=0, keepdims=True)

    gb = i * _BB + jax.lax.broadcasted_iota(jnp.int32, (1, _BB), 1)
    fidx_ref[...] = idx * 32768 + (gb // 128) * 256 + (gb % 128)


def _rel_gather(B):
    mesh = plsc.VectorSubcoreMesh(core_axis_name="c", subcore_axis_name="s")
    rows = B // 128
    rows_per_w = rows // 32

    @functools.partial(
        pl.kernel, mesh=mesh,
        out_type=jax.ShapeDtypeStruct((rows, 128), jnp.int32),
        scratch_types=[
            pltpu.VMEM((128,), jnp.int32),
            pltpu.VMEM((128,), jnp.int32),
            pltpu.SemaphoreType.DMA,
        ],
    )
    def k(flat_hbm, fidx_hbm, out_hbm, idx_v, rel_v, sem):
        wid = lax.axis_index("s") * 2 + lax.axis_index("c")
        for j in range(rows_per_w):
            r = wid * rows_per_w + j
            pltpu.sync_copy(fidx_hbm.at[r], idx_v)
            pltpu.async_copy(flat_hbm.at[idx_v], rel_v, sem).wait()
            pltpu.sync_copy(rel_v, out_hbm.at[r])

    return k


def kernel(prelim_scores, actions_id, u):
    B, N = prelim_scores.shape
    ps_t = prelim_scores.T
    u_t = u.T
    acts_t = jnp.transpose(actions_id, (1, 2, 0))

    col_spec = pl.BlockSpec((N, _BB), lambda i: (0, i))
    act_spec = pl.BlockSpec(memory_space=pltpu.MemorySpace.HBM)
    one_spec = pl.BlockSpec((1, _BB), lambda i: (0, i))
    outs = pl.pallas_call(
        _body,
        grid=(B // _BB,),
        in_specs=[col_spec, act_spec, col_spec],
        out_specs=[col_spec, one_spec, one_spec, one_spec, one_spec],
        out_shape=[
            jax.ShapeDtypeStruct((N, B), jnp.float32),
            jax.ShapeDtypeStruct((1, B), jnp.float32),
            jax.ShapeDtypeStruct((1, B), jnp.int32),
            jax.ShapeDtypeStruct((1, B), jnp.int32),
            jax.ShapeDtypeStruct((1, B), jnp.int32),
        ],
        scratch_shapes=[
            pltpu.VMEM((N, _BB), jnp.int32),
            pltpu.SemaphoreType.DMA,
        ],
    )(ps_t, acts_t, u_t)
    logits_t, loss, aid, ent_o, fidx = outs

    flat = jnp.transpose(
        actions_id.reshape(B // 128, 128, N, 2), (2, 0, 3, 1)
    ).reshape(N * B * 2)
    fidx2 = fidx.reshape(B // 128, 128)
    rel2 = _rel_gather(B)(flat, fidx2)
    return (loss[0], logits_t.T, aid[0], ent_o[0], rel2.reshape(B))


# speedup vs baseline: 2.0129x; 1.4442x over previous
import functools

import jax
import jax.numpy as jnp
from jax import lax
from jax.experimental import pallas as pl
from jax.experimental.pallas import tpu as pltpu
from jax.experimental.pallas import tpu_sc as plsc

_PAD = 0
_NEG = -99999.0
_BB = 1024


def _body(ps_ref, act_ref, u_ref,
          logits_ref, loss_ref, aid_ref, ent_o_ref, fidx_ref,
          ent_s, sem):
    n = ps_ref.shape[0]
    i = pl.program_id(0)

    def ent_copy(blk, slot):
        return pltpu.make_async_copy(
            act_ref.at[:, 1, pl.ds(blk * _BB, _BB)], ent_s.at[slot],
            sem.at[slot])

    @pl.when(i == 0)
    def _():
        ent_copy(0, 0).start()

    @pl.when(i + 1 < pl.num_programs(0))
    def _():
        ent_copy(i + 1, (i + 1) % 2).start()

    ps = ps_ref[...]
    u = u_ref[...]
    gumbel = -jnp.log(-jnp.log(u))
    ent_copy(i, i % 2).wait()
    ent = ent_s[i % 2]

    scores = jnp.where(ent == _PAD, _NEG, ps)
    m = jnp.max(scores, axis=0, keepdims=True)
    shifted = scores - m
    sumexp = jnp.sum(jnp.exp(shifted), axis=0, keepdims=True)
    logits = shifted - jnp.log(sumexp)
    logits_ref[...] = logits

    y = logits + gumbel
    ymax = jnp.max(y, axis=0, keepdims=True)
    n_iota = jax.lax.broadcasted_iota(jnp.int32, y.shape, 0)
    idx = jnp.min(jnp.where(y == ymax, n_iota, jnp.int32(n)),
                  axis=0, keepdims=True)
    aid_ref[...] = idx

    sel = n_iota == idx
    loss_ref[...] = -jnp.sum(jnp.where(sel, logits, 0.0), axis=0, keepdims=True)
    ent_o_ref[...] = jnp.sum(jnp.where(sel, ent, 0), axis=0, keepdims=True)

    gb = i * _BB + jax.lax.broadcasted_iota(jnp.int32, (1, _BB), 1)
    fidx_ref[...] = idx * 32768 + (gb // 128) * 256 + (gb % 128)


def _rel_gather(B):
    mesh = plsc.VectorSubcoreMesh(core_axis_name="c", subcore_axis_name="s")
    rows = B // 128
    rows_per_w = rows // 32

    @functools.partial(
        pl.kernel, mesh=mesh,
        out_type=jax.ShapeDtypeStruct((rows, 128), jnp.int32),
        scratch_types=[
            pltpu.VMEM((128,), jnp.int32),
            pltpu.VMEM((128,), jnp.int32),
            pltpu.SemaphoreType.DMA,
        ],
    )
    def k(flat_hbm, fidx_hbm, out_hbm, idx_v, rel_v, sem):
        wid = lax.axis_index("s") * 2 + lax.axis_index("c")
        for j in range(rows_per_w):
            r = wid * rows_per_w + j
            pltpu.sync_copy(fidx_hbm.at[r], idx_v)
            pltpu.async_copy(flat_hbm.at[idx_v], rel_v, sem).wait()
            pltpu.sync_copy(rel_v, out_hbm.at[r])

    return k


def kernel(prelim_scores, actions_id, u):
    B, N = prelim_scores.shape
    ps_t = prelim_scores.T
    u_t = u.T
    acts_t = jnp.transpose(actions_id, (1, 2, 0))

    col_spec = pl.BlockSpec((N, _BB), lambda i: (0, i))
    act_spec = pl.BlockSpec(memory_space=pltpu.MemorySpace.HBM)
    one_spec = pl.BlockSpec((1, _BB), lambda i: (0, i))
    outs = pl.pallas_call(
        _body,
        grid=(B // _BB,),
        in_specs=[col_spec, act_spec, col_spec],
        out_specs=[col_spec, one_spec, one_spec, one_spec, one_spec],
        out_shape=[
            jax.ShapeDtypeStruct((N, B), jnp.float32),
            jax.ShapeDtypeStruct((1, B), jnp.float32),
            jax.ShapeDtypeStruct((1, B), jnp.int32),
            jax.ShapeDtypeStruct((1, B), jnp.int32),
            jax.ShapeDtypeStruct((1, B), jnp.int32),
        ],
        scratch_shapes=[
            pltpu.VMEM((2, N, _BB), jnp.int32),
            pltpu.SemaphoreType.DMA((2,)),
        ],
    )(ps_t, acts_t, u_t)
    logits_t, loss, aid, ent_o, fidx = outs

    flat = jnp.transpose(
        actions_id.reshape(B // 128, 128, N, 2), (2, 0, 3, 1)
    ).reshape(N * B * 2)
    fidx2 = fidx.reshape(B // 128, 128)
    rel2 = _rel_gather(B)(flat, fidx2)
    return (loss[0], logits_t.T, aid[0], ent_o[0], rel2.reshape(B))


# speedup vs baseline: 2.0675x; 1.0271x over previous
import functools

import jax
import jax.numpy as jnp
from jax import lax
from jax.experimental import pallas as pl
from jax.experimental.pallas import tpu as pltpu
from jax.experimental.pallas import tpu_sc as plsc

_PAD = 0
_NEG = -99999.0
_BB = 1024


def _body(ps_ref, act_ref, u_ref,
          logits_ref, loss_ref, aid_ref, ent_o_ref, fidx_ref,
          ent_s, sem):
    n = ps_ref.shape[0]
    i = pl.program_id(0)

    def ent_copy(blk, slot):
        return pltpu.make_async_copy(
            act_ref.at[:, 1, pl.ds(blk * _BB, _BB)], ent_s.at[slot],
            sem.at[slot])

    @pl.when(i == 0)
    def _():
        ent_copy(0, 0).start()

    @pl.when(i + 1 < pl.num_programs(0))
    def _():
        ent_copy(i + 1, (i + 1) % 2).start()

    ps = ps_ref[...]
    u = u_ref[...]
    gumbel = -jnp.log(-jnp.log(u))
    ent_copy(i, i % 2).wait()
    ent = ent_s[i % 2]

    scores = jnp.where(ent == _PAD, _NEG, ps)
    m = jnp.max(scores, axis=0, keepdims=True)
    shifted = scores - m
    sumexp = jnp.sum(jnp.exp(shifted), axis=0, keepdims=True)
    logits = shifted - jnp.log(sumexp)
    logits_ref[...] = logits

    y = logits + gumbel
    ymax = jnp.max(y, axis=0, keepdims=True)
    n_iota = jax.lax.broadcasted_iota(jnp.int32, y.shape, 0)
    idx = jnp.min(jnp.where(y == ymax, n_iota, jnp.int32(n)),
                  axis=0, keepdims=True)
    aid_ref[...] = idx

    sel = n_iota == idx
    loss_ref[...] = -jnp.sum(jnp.where(sel, logits, 0.0), axis=0, keepdims=True)
    ent_o_ref[...] = jnp.sum(jnp.where(sel, ent, 0), axis=0, keepdims=True)

    gb = i * _BB + jax.lax.broadcasted_iota(jnp.int32, (1, _BB), 1)
    fidx_ref[...] = idx * 32768 + (gb // 128) * 256 + (gb % 128)


def _rel_gather(B):
    mesh = plsc.VectorSubcoreMesh(core_axis_name="c", subcore_axis_name="s")
    rows = B // 128
    rows_per_w = rows // 32

    @functools.partial(
        pl.kernel, mesh=mesh,
        out_type=jax.ShapeDtypeStruct((rows, 128), jnp.int32),
        scratch_types=[
            pltpu.VMEM((rows // 32, 128), jnp.int32),
            pltpu.VMEM((rows // 32, 128), jnp.int32),
            pltpu.SemaphoreType.DMA,
        ],
    )
    def k(flat_hbm, fidx_hbm, out_hbm, idx_v, rel_v, sem):
        wid = lax.axis_index("s") * 2 + lax.axis_index("c")
        r0 = wid * rows_per_w
        pltpu.sync_copy(fidx_hbm.at[pl.ds(r0, rows_per_w)], idx_v)
        cps = [pltpu.async_copy(flat_hbm.at[idx_v.at[j]], rel_v.at[j], sem)
               for j in range(rows_per_w)]
        for cp in cps:
            cp.wait()
        pltpu.sync_copy(rel_v, out_hbm.at[pl.ds(r0, rows_per_w)])

    return k


def kernel(prelim_scores, actions_id, u):
    B, N = prelim_scores.shape
    ps_t = prelim_scores.T
    u_t = u.T
    acts_t = jnp.transpose(actions_id, (1, 2, 0))

    col_spec = pl.BlockSpec((N, _BB), lambda i: (0, i))
    act_spec = pl.BlockSpec(memory_space=pltpu.MemorySpace.HBM)
    one_spec = pl.BlockSpec((1, _BB), lambda i: (0, i))
    outs = pl.pallas_call(
        _body,
        grid=(B // _BB,),
        in_specs=[col_spec, act_spec, col_spec],
        out_specs=[col_spec, one_spec, one_spec, one_spec, one_spec],
        out_shape=[
            jax.ShapeDtypeStruct((N, B), jnp.float32),
            jax.ShapeDtypeStruct((1, B), jnp.float32),
            jax.ShapeDtypeStruct((1, B), jnp.int32),
            jax.ShapeDtypeStruct((1, B), jnp.int32),
            jax.ShapeDtypeStruct((1, B), jnp.int32),
        ],
        scratch_shapes=[
            pltpu.VMEM((2, N, _BB), jnp.int32),
            pltpu.SemaphoreType.DMA((2,)),
        ],
    )(ps_t, acts_t, u_t)
    logits_t, loss, aid, ent_o, fidx = outs

    flat = jnp.transpose(
        actions_id.reshape(B // 128, 128, N, 2), (2, 0, 3, 1)
    ).reshape(N * B * 2)
    fidx2 = fidx.reshape(B // 128, 128)
    rel2 = _rel_gather(B)(flat, fidx2)
    return (loss[0], logits_t.T, aid[0], ent_o[0], rel2.reshape(B))
